# baseline (speedup 1.0000x reference)
import functools

import jax
import jax.numpy as jnp
from jax import lax
from jax.experimental import pallas as pl
from jax.experimental.pallas import tpu as pltpu
from jax.experimental.pallas import tpu_sc as plsc

B, H, Q, D = 16, 16, 16, 128
TOT = B * H * Q * D
NW = 32
PER = TOT // NW

_mesh = plsc.VectorSubcoreMesh(core_axis_name="c", subcore_axis_name="s")


@functools.partial(
    pl.kernel,
    out_type=(
        jax.ShapeDtypeStruct((NW, PER), jnp.float32),
        jax.ShapeDtypeStruct((NW, PER), jnp.float32),
    ),
    mesh=_mesh,
    scratch_types=[
        pltpu.VMEM((PER,), jnp.float32),
        pltpu.VMEM((PER,), jnp.float32),
        pltpu.SemaphoreType.DMA,
        pltpu.SemaphoreType.DMA,
    ],
)
def _scatter_copy(k_hbm, v_hbm, ko_hbm, vo_hbm, kbuf, vbuf, ksem, vsem):
    wid = lax.axis_index("s") * 2 + lax.axis_index("c")
    ck = pltpu.async_copy(k_hbm.at[wid], kbuf, ksem)
    cv = pltpu.async_copy(v_hbm.at[wid], vbuf, vsem)
    ck.wait()
    ck2 = pltpu.async_copy(kbuf, ko_hbm.at[wid], ksem)
    cv.wait()
    cv2 = pltpu.async_copy(vbuf, vo_hbm.at[wid], vsem)
    ck2.wait()
    cv2.wait()


def kernel(k_val, v_val, k_cache, v_cache):
    ko, vo = _scatter_copy(k_val.reshape(NW, PER), v_val.reshape(NW, PER))
    return (ko.reshape(B, H, Q, D), vo.reshape(B, H, Q, D))


# speedup vs baseline: 1.1447x; 1.1447x over previous
import functools

import jax
import jax.numpy as jnp
from jax import lax
from jax.experimental import pallas as pl
from jax.experimental.pallas import tpu as pltpu
from jax.experimental.pallas import tpu_sc as plsc

B, H, Q, D = 16, 16, 16, 128
TOT = B * H * Q * D
NW = 32
PER = TOT // NW

_mesh = plsc.VectorSubcoreMesh(core_axis_name="c", subcore_axis_name="s")


@functools.partial(
    pl.kernel,
    out_type=jax.ShapeDtypeStruct((NW, PER), jnp.float32),
    mesh=_mesh,
    scratch_types=[
        pltpu.VMEM((PER,), jnp.float32),
        pltpu.SemaphoreType.DMA,
    ],
)
def _scatter_copy_one(k_hbm, ko_hbm, kbuf, ksem):
    wid = lax.axis_index("s") * 2 + lax.axis_index("c")
    pltpu.async_copy(k_hbm.at[wid], kbuf, ksem).wait()
    pltpu.async_copy(kbuf, ko_hbm.at[wid], ksem).wait()


def _tc_copy_body(x_ref, o_ref):
    o_ref[...] = x_ref[...]


_tc_copy = pl.pallas_call(
    _tc_copy_body,
    out_shape=jax.ShapeDtypeStruct((B * H * Q, D), jnp.float32),
)


def kernel(k_val, v_val, k_cache, v_cache):
    ko = _scatter_copy_one(k_val.reshape(NW, PER))
    vo = _tc_copy(v_val.reshape(B * H * Q, D))
    return (ko.reshape(B, H, Q, D), vo.reshape(B, H, Q, D))
